# baseline (speedup 1.0000x reference)
import functools

import jax
import jax.numpy as jnp
from jax import lax
from jax.experimental import pallas as pl
from jax.experimental.pallas import tpu as pltpu
from jax.experimental.pallas import tpu_sc as plsc

N_NODES = 10000
N_EDGES = 320000
D = 128

NC = 2
NS = 16
NW = NC * NS
EPW = N_EDGES // NW
CHUNK = 80
NCHUNK = EPW // CHUNK
NBUF = 4
IRING = 8
LA = 3
LAI = 6
UNROLL = 8
NITER = 128 // UNROLL
ROWS_PER_TILE = 624
ROWS_REM = N_NODES - NS * ROWS_PER_TILE

_mesh = plsc.VectorSubcoreMesh(core_axis_name="c", subcore_axis_name="s")


@functools.partial(
    pl.kernel,
    mesh=_mesh,
    out_type=jax.ShapeDtypeStruct((NC, N_NODES, D), jnp.float32),
    scratch_types=(
        [pltpu.VMEM((IRING, CHUNK), jnp.int32)] * 2
        + [pltpu.VMEM((CHUNK, D), jnp.float32)] * NBUF
        + [pltpu.VMEM_SHARED((N_NODES, D), jnp.float32)]
        + [pltpu.SemaphoreType.DMA] * (2 * NBUF + IRING)
    ),
)
def _sc_aggregate(x_hbm, edges_hbm, out_hbm, si, di, *rest):
    rows = rest[:NBUF]
    agg = rest[NBUF]
    gsem = rest[NBUF + 1:NBUF + 1 + NBUF]
    ssem = rest[NBUF + 1 + NBUF:NBUF + 1 + 2 * NBUF]
    isem = rest[NBUF + 1 + 2 * NBUF:]
    c = lax.axis_index("c")
    s = lax.axis_index("s")
    wid = c * NS + s
    base = wid * EPW
    row0 = s * ROWS_PER_TILE

    def fire_idx(ch, slot):
        off = pl.multiple_of(base + ch * CHUNK, 8)
        pltpu.async_copy(edges_hbm.at[pl.ds(off, CHUNK)], si.at[slot],
                         isem[slot])
        pltpu.async_copy(edges_hbm.at[pl.ds(N_EDGES + off, CHUNK)],
                         di.at[slot], isem[slot])

    def wait_idx(slot):
        pltpu.make_async_copy(edges_hbm.at[pl.ds(0, CHUNK)], si.at[slot],
                              isem[slot]).wait()
        pltpu.make_async_copy(edges_hbm.at[pl.ds(0, CHUNK)], di.at[slot],
                              isem[slot]).wait()

    def wait_gather(slot):
        pltpu.make_async_copy(x_hbm.at[si.at[0]], rows[slot],
                              gsem[slot]).wait()

    def wait_scatter(slot):
        pltpu.make_async_copy(rows[slot], agg.at[di.at[0]],
                              ssem[slot]).wait()

    for j in range(LAI):
        fire_idx(j, j)
    for u in range(LA):
        wait_idx(u)
        pltpu.async_copy(x_hbm.at[si.at[u]], rows[u], gsem[u])

    @pl.when(c == 0)
    def _():
        pltpu.sync_copy(x_hbm.at[pl.ds(row0, ROWS_PER_TILE)],
                        agg.at[pl.ds(row0, ROWS_PER_TILE)])

        @pl.when(s == NS - 1)
        def _():
            pltpu.sync_copy(x_hbm.at[pl.ds(NS * ROWS_PER_TILE, ROWS_REM)],
                            agg.at[pl.ds(NS * ROWS_PER_TILE, ROWS_REM)])

    @pl.when(c != 0)
    def _():
        def zero_row(r, carry):
            for j in range(D // 16):
                rows[NBUF - 1][r, pl.ds(16 * j, 16)] = jnp.zeros(
                    (16,), jnp.float32)
            return carry

        lax.fori_loop(0, CHUNK, zero_row, 0)
        for j in range(ROWS_PER_TILE // CHUNK):
            pltpu.sync_copy(rows[NBUF - 1],
                            agg.at[pl.ds(row0 + j * CHUNK, CHUNK)])
        rem = ROWS_PER_TILE % CHUNK
        pltpu.sync_copy(rows[NBUF - 1].at[pl.ds(0, rem)],
                        agg.at[pl.ds(row0 + ROWS_PER_TILE - rem, rem)])

        @pl.when(s == NS - 1)
        def _():
            pltpu.sync_copy(rows[NBUF - 1].at[pl.ds(0, ROWS_REM)],
                            agg.at[pl.ds(NS * ROWS_PER_TILE, ROWS_REM)])

    plsc.subcore_barrier()

    def body(k, carry):
        for u in range(UNROLL):
            g = k * UNROLL + u
            b = u % NBUF
            gp = g + LA
            bp = (u + LA) % NBUF
            bgi = (u + LA) % IRING
            gi = g + LAI
            bi = (u + LAI) % IRING

            @pl.when((gp >= NBUF) & (gp < NCHUNK))
            def _():
                wait_scatter(bp)

            @pl.when(gi < NCHUNK)
            def _():
                fire_idx(gi, bi)

            @pl.when(gp < NCHUNK)
            def _():
                wait_idx(bgi)
                pltpu.async_copy(x_hbm.at[si.at[bgi]], rows[bp], gsem[bp])

            @pl.when(g < NCHUNK)
            def _():
                wait_gather(b)
                pltpu.async_copy(rows[b], agg.at[di.at[u]], ssem[b], add=True)
        return carry

    lax.fori_loop(0, NITER, body, 0)

    for b in range(NBUF):
        wait_scatter(b)

    plsc.subcore_barrier()
    pltpu.sync_copy(agg.at[pl.ds(row0, ROWS_PER_TILE)],
                    out_hbm.at[c, pl.ds(row0, ROWS_PER_TILE)])

    @pl.when(s == NS - 1)
    def _():
        pltpu.sync_copy(agg.at[pl.ds(NS * ROWS_PER_TILE, ROWS_REM)],
                        out_hbm.at[c, pl.ds(NS * ROWS_PER_TILE, ROWS_REM)])


def _tc_mlp(p_ref, w_ref, b_ref, o_ref):
    h = p_ref[0] + p_ref[1]
    y = lax.dot_general(h, w_ref[...], (((1,), (1,)), ((), ())),
                        preferred_element_type=jnp.float32)
    o_ref[...] = jnp.maximum(y + b_ref[...], 0.0)


_BR = 10000


def kernel(x, edge_index, W, b):
    edges = edge_index.reshape(2 * N_EDGES)
    partial = _sc_aggregate(x, edges)

    out = pl.pallas_call(
        _tc_mlp,
        grid=(N_NODES // _BR,),
        in_specs=[
            pl.BlockSpec((NC, _BR, D), lambda r: (0, r, 0)),
            pl.BlockSpec((D, D), lambda r: (0, 0)),
            pl.BlockSpec((1, D), lambda r: (0, 0)),
        ],
        out_specs=pl.BlockSpec((_BR, D), lambda r: (r, 0)),
        out_shape=jax.ShapeDtypeStruct((N_NODES, D), jnp.float32),
    )(partial, W, b.reshape(1, D))
    return out
